# baseline (speedup 1.0000x reference)
import functools

import jax
import jax.numpy as jnp
from jax import lax
from jax.experimental import pallas as pl
from jax.experimental.pallas import tpu as pltpu
from jax.experimental.pallas import tpu_sc as plsc

N = 10000
EDGES = 320000
F = 128

NC, NS = 2, 16
NW = NC * NS
EW = EDGES // NW
K = 64
CB = 16
NBW = 160
NCH = NBW // CB
EWP = NBW * K
RPT = 624
RTAIL = N - NS * RPT

BN = 1000


def _pre_body(x_ref, w_ref, a2_ref, h_ref, al2_ref):
    h = jnp.dot(x_ref[...], w_ref[...], preferred_element_type=jnp.float32)
    h_ref[...] = h
    al2_ref[...] = jnp.dot(h, a2_ref[...], preferred_element_type=jnp.float32)


def _pre(x, W, a2):
    return pl.pallas_call(
        _pre_body,
        grid=(N // BN,),
        in_specs=[pl.BlockSpec((BN, F), lambda i: (i, 0)),
                  pl.BlockSpec((F, F), lambda i: (0, 0)),
                  pl.BlockSpec((F, 2), lambda i: (0, 0))],
        out_specs=[pl.BlockSpec((BN, F), lambda i: (i, 0)),
                   pl.BlockSpec((BN, 2), lambda i: (i, 0))],
        out_shape=[jax.ShapeDtypeStruct((N, F), jnp.float32),
                   jax.ShapeDtypeStruct((N, 2), jnp.float32)],
    )(x, W, a2)


def _mid_body(p0_ref, p1_ref, d0_ref, d1_ref, hp_ref, b_ref, w_ref, a2_ref,
              hn_ref, al2_ref):
    acc = p0_ref[...] + p1_ref[...] + hp_ref[...]
    inv = 1.0 / (d0_ref[...] + d1_ref[...] + 1.0 + 1e-16)
    o = acc * inv + b_ref[...]
    act = jnp.where(o > 0, o, jnp.exp(o) - 1.0)
    hn = jnp.dot(act, w_ref[...], preferred_element_type=jnp.float32)
    hn_ref[...] = hn
    al2_ref[...] = jnp.dot(hn, a2_ref[...], preferred_element_type=jnp.float32)


def _mid(p0, p1, d0, d1, hp, b, W, a2):
    return pl.pallas_call(
        _mid_body,
        grid=(N // BN,),
        in_specs=[pl.BlockSpec((BN, F), lambda i: (i, 0)),
                  pl.BlockSpec((BN, F), lambda i: (i, 0)),
                  pl.BlockSpec((BN, 1), lambda i: (i, 0)),
                  pl.BlockSpec((BN, 1), lambda i: (i, 0)),
                  pl.BlockSpec((BN, F), lambda i: (i, 0)),
                  pl.BlockSpec((1, F), lambda i: (0, 0)),
                  pl.BlockSpec((F, F), lambda i: (0, 0)),
                  pl.BlockSpec((F, 2), lambda i: (0, 0))],
        out_specs=[pl.BlockSpec((BN, F), lambda i: (i, 0)),
                   pl.BlockSpec((BN, 2), lambda i: (i, 0))],
        out_shape=[jax.ShapeDtypeStruct((N, F), jnp.float32),
                   jax.ShapeDtypeStruct((N, 2), jnp.float32)],
    )(p0, p1, d0, d1, hp, b, W, a2)


def _fin_body(p0_ref, p1_ref, d0_ref, d1_ref, hp_ref, b_ref, out_ref):
    acc = p0_ref[...] + p1_ref[...] + hp_ref[...]
    inv = 1.0 / (d0_ref[...] + d1_ref[...] + 1.0 + 1e-16)
    o = acc * inv + b_ref[...]
    m = jnp.max(o, axis=-1, keepdims=True)
    z = o - m
    out_ref[...] = z - jnp.log(jnp.sum(jnp.exp(z), axis=-1, keepdims=True))


def _fin(p0, p1, d0, d1, hp, b):
    return pl.pallas_call(
        _fin_body,
        grid=(N // BN,),
        in_specs=[pl.BlockSpec((BN, F), lambda i: (i, 0)),
                  pl.BlockSpec((BN, F), lambda i: (i, 0)),
                  pl.BlockSpec((BN, 1), lambda i: (i, 0)),
                  pl.BlockSpec((BN, 1), lambda i: (i, 0)),
                  pl.BlockSpec((BN, F), lambda i: (i, 0)),
                  pl.BlockSpec((1, F), lambda i: (0, 0))],
        out_specs=pl.BlockSpec((BN, F), lambda i: (i, 0)),
        out_shape=jax.ShapeDtypeStruct((N, F), jnp.float32),
    )(p0, p1, d0, d1, hp, b)


_MESH = plsc.VectorSubcoreMesh(core_axis_name="c", subcore_axis_name="s")


@functools.partial(
    pl.kernel,
    out_type=(jax.ShapeDtypeStruct((N, F), jnp.float32),
              jax.ShapeDtypeStruct((N, F), jnp.float32),
              jax.ShapeDtypeStruct((640 * NS,), jnp.float32),
              jax.ShapeDtypeStruct((640 * NS,), jnp.float32)),
    mesh=_MESH,
    compiler_params=pltpu.CompilerParams(needs_layout_passes=False),
    scratch_types=[
        pltpu.VMEM_SHARED((N, F), jnp.float32),
        pltpu.VMEM_SHARED((640 * NS,), jnp.float32),
        pltpu.VMEM((N,), jnp.float32),
        pltpu.VMEM((N,), jnp.float32),
        pltpu.VMEM((2, CB, K), jnp.int32),
        pltpu.VMEM((3, K), jnp.float32),
        pltpu.VMEM((K, F), jnp.float32),
        pltpu.VMEM((K, F), jnp.float32),
        pltpu.VMEM((K, F), jnp.float32),
        pltpu.VMEM((320,), jnp.float32),
        pltpu.SemaphoreType.DMA,
        pltpu.SemaphoreType.DMA,
        pltpu.SemaphoreType.DMA,
        pltpu.SemaphoreType.DMA,
        pltpu.SemaphoreType.DMA,
        pltpu.SemaphoreType.DMA,
    ],
)
def _edge_pass(adjp, h, als, ald,
               p0, p1, dn0, dn1,
               accum, dnacc, as_l, ad_l, cbuf, exring,
               rows0, rows1, rows2, zb,
               semg0, semg1, semg2, sems0, sems1, sems2):
    cid = lax.axis_index("c")
    sid = lax.axis_index("s")
    wid = cid * NS + sid

    pltpu.sync_copy(als, as_l)
    pltpu.sync_copy(ald, ad_l)

    z16 = jnp.zeros((16,), jnp.float32)

    def _z1(i, c):
        zb[pl.ds(i * 16, 16)] = z16
        return c
    lax.fori_loop(0, 320 // 16, _z1, 0)

    def _zr(k, c):
        for g in range(F // 16):
            rows0[k, pl.ds(g * 16, 16)] = z16
        return c
    lax.fori_loop(0, K, _zr, 0)

    pltpu.sync_copy(zb, dnacc.at[pl.ds(sid * 640, 320)])
    pltpu.sync_copy(zb, dnacc.at[pl.ds(sid * 640 + 320, 320)])
    r0 = sid * RPT
    nfull = RPT // K
    rem = RPT - nfull * K

    def _za(i, c):
        pltpu.sync_copy(rows0, accum.at[pl.ds(r0 + i * K, K)])
        return c
    lax.fori_loop(0, nfull, _za, 0)
    pltpu.sync_copy(rows0.at[pl.ds(0, rem)], accum.at[pl.ds(r0 + nfull * K, rem)])

    @pl.when(sid == NS - 1)
    def _():
        pltpu.sync_copy(rows0.at[pl.ds(0, RTAIL)],
                        accum.at[pl.ds(NS * RPT, RTAIL)])
    plsc.subcore_barrier()

    wrow = wid * NBW
    rowss = (rows0, rows1, rows2)
    semgs = (semg0, semg1, semg2)
    semss = (sems0, sems1, sems2)

    def _fire_gather(jb, b):
        pltpu.async_copy(h.at[cbuf.at[0, jb]], rowss[b], semgs[b])

    def _wait_gather(jb, b):
        pltpu.make_async_copy(h.at[cbuf.at[0, jb]], rowss[b], semgs[b]).wait()

    def _fire_scatter(jb, b):
        pltpu.async_copy(exring.at[b], dnacc.at[cbuf.at[1, jb]], semss[b], add=True)
        pltpu.async_copy(rowss[b], accum.at[cbuf.at[1, jb]], semss[b], add=True)

    def _drain_scatter(jb, b):
        pltpu.make_async_copy(exring.at[b], dnacc.at[cbuf.at[1, jb]], semss[b]).wait()
        pltpu.make_async_copy(rowss[b], accum.at[cbuf.at[1, jb]], semss[b]).wait()

    iota16 = lax.broadcasted_iota(jnp.int32, (16,), 0)

    def _scalar_pass(b, jb, base_lid):
        def _grp(j, c2):
            s16 = cbuf[0, jb, pl.ds(j * 16, 16)]
            d16 = cbuf[1, jb, pl.ds(j * 16, 16)]
            a = plsc.load_gather(as_l, [s16]) + plsc.load_gather(ad_l, [d16])
            sl = plsc.load_gather(as_l, [d16]) + plsc.load_gather(ad_l, [d16])
            e = jnp.where(a >= 0, a, 0.2 * a)
            es = jnp.where(sl >= 0, sl, 0.2 * sl)
            ex = jnp.exp(e - es)
            lid = base_lid + j * 16 + iota16
            exring[b, pl.ds(j * 16, 16)] = jnp.where(lid < EW, ex, 0.0)
            return c2
        lax.fori_loop(0, K // 16, _grp, 0)

    def _scale(b):
        rows = rowss[b]

        def _srow(j, c2):
            ex16 = exring[b, pl.ds(j * 16, 16)]
            bk = j * 16
            for l in range(16):
                s = ex16[l]
                for g in range(F // 16):
                    rows[bk + l, pl.ds(g * 16, 16)] = rows[bk + l, pl.ds(g * 16, 16)] * s
            return c2
        lax.fori_loop(0, K // 16, _srow, 0)

    def _chunk(c, carry):
        @pl.when(c >= 1)
        def _():
            _drain_scatter(CB - 2, (CB - 2) % 3)
            _drain_scatter(CB - 1, (CB - 1) % 3)
        pltpu.sync_copy(adjp.at[:, pl.ds(wrow + c * CB, CB), :], cbuf)
        _fire_gather(0, 0)
        base_c = c * (CB * K)
        for jb in range(CB):
            b = jb % 3
            if jb >= 2:
                _drain_scatter(jb - 2, (jb - 2) % 3)
            _scalar_pass(b, jb, base_c + jb * K)
            if jb < CB - 1:
                _fire_gather(jb + 1, (jb + 1) % 3)
            _wait_gather(jb, b)
            _scale(b)
            _fire_scatter(jb, b)
        return carry
    lax.fori_loop(0, NCH, _chunk, 0)

    _drain_scatter(CB - 2, (CB - 2) % 3)
    _drain_scatter(CB - 1, (CB - 1) % 3)

    plsc.subcore_barrier()

    def _copy_out(p, dn):
        def _co(i, c):
            pltpu.sync_copy(accum.at[pl.ds(r0 + i * K, K)], rows0)
            pltpu.sync_copy(rows0, p.at[pl.ds(r0 + i * K, K)])
            return c
        lax.fori_loop(0, nfull, _co, 0)
        pltpu.sync_copy(accum.at[pl.ds(r0 + nfull * K, rem)], rows0.at[pl.ds(0, rem)])
        pltpu.sync_copy(rows0.at[pl.ds(0, rem)], p.at[pl.ds(r0 + nfull * K, rem)])

        @pl.when(sid == NS - 1)
        def _():
            pltpu.sync_copy(accum.at[pl.ds(NS * RPT, RTAIL)],
                            rows0.at[pl.ds(0, RTAIL)])
            pltpu.sync_copy(rows0.at[pl.ds(0, RTAIL)],
                            p.at[pl.ds(NS * RPT, RTAIL)])
        for q in range(2):
            pltpu.sync_copy(dnacc.at[pl.ds(sid * 640 + q * 320, 320)], zb)
            pltpu.sync_copy(zb, dn.at[pl.ds(sid * 640 + q * 320, 320)])

    @pl.when(cid == 0)
    def _():
        _copy_out(p0, dn0)

    @pl.when(cid == 1)
    def _():
        _copy_out(p1, dn1)


def _layer_sc(adjp, h, al2):
    p0, p1, dn0, dn1 = _edge_pass(adjp, h, al2[:, 0], al2[:, 1])
    d0 = dn0[:N].reshape(N, 1)
    d1 = dn1[:N].reshape(N, 1)
    return p0, p1, d0, d1


def kernel(x, adj_t, W1, a_src1, a_dst1, b1, W2, a_src2, a_dst2, b2,
           W3, a_src3, a_dst3, b3):
    adjp = jnp.pad(adj_t.reshape(2, NW, EW),
                   ((0, 0), (0, 0), (0, EWP - EW))).reshape(2, NW * NBW, K)
    a21 = jnp.stack([a_src1, a_dst1], axis=1)
    a22 = jnp.stack([a_src2, a_dst2], axis=1)
    a23 = jnp.stack([a_src3, a_dst3], axis=1)

    h1, al21 = _pre(x, W1, a21)
    p0, p1, d0, d1 = _layer_sc(adjp, h1, al21)
    h2, al22 = _mid(p0, p1, d0, d1, h1, b1.reshape(1, F), W2, a22)
    p0, p1, d0, d1 = _layer_sc(adjp, h2, al22)
    h3, al23 = _mid(p0, p1, d0, d1, h2, b2.reshape(1, F), W3, a23)
    p0, p1, d0, d1 = _layer_sc(adjp, h3, al23)
    return _fin(p0, p1, d0, d1, h3, b3.reshape(1, F))
